# baseline (speedup 1.0000x reference)
import functools

import jax
import jax.numpy as jnp
from jax import lax
from jax.experimental import pallas as pl
from jax.experimental.pallas import tpu as pltpu
from jax.experimental.pallas import tpu_sc as plsc

TEMPERATURE = 0.5
BIAS = 0.0001

NC = 2
NS = 16
NW = NC * NS
LANES = 16

GRID = 10


def _tc_body(x_ref, rel_ref, g_ref,
             sw1_ref, sb1_ref, sw2_ref, sb2_ref,
             dw1_ref, db1_ref, dw2_ref, db2_ref,
             ew1_ref, eb1_ref, ew2_ref, eb2_ref,
             scores_ref, ge_ref):
    x = x_ref[...]
    hs = jnp.maximum(
        jnp.dot(x, sw1_ref[...], preferred_element_type=jnp.float32)
        + sb1_ref[...], 0.0)
    ss = jnp.dot(hs, sw2_ref[...], preferred_element_type=jnp.float32) + sb2_ref[...]
    hd = jnp.maximum(
        jnp.dot(x, dw1_ref[...], preferred_element_type=jnp.float32)
        + db1_ref[...], 0.0)
    sd = jnp.dot(hd, dw2_ref[...], preferred_element_type=jnp.float32) + db2_ref[...]
    scores_ref[...] = jnp.concatenate([ss, sd], axis=1)

    he = jnp.maximum(
        jnp.dot(rel_ref[...], ew1_ref[...], preferred_element_type=jnp.float32)
        + eb1_ref[...], 0.0)
    ge_ref[...] = (
        jnp.dot(he, ew2_ref[...], preferred_element_type=jnp.float32)
        + eb2_ref[...] + g_ref[...])


def _tc_mlps(node_emb, relation_emb, g2,
             src_w1, src_b1, src_w2, src_b2,
             dst_w1, dst_b1, dst_w2, dst_b2,
             edge_w1, edge_b1, edge_w2, edge_b2):
    n, d = node_emb.shape
    e, de = relation_emb.shape
    h = src_w1.shape[1]
    nblk = n // GRID
    eblk = e // GRID
    full = lambda i: (0, 0)
    full1 = lambda i: (0,)
    return pl.pallas_call(
        _tc_body,
        grid=(GRID,),
        in_specs=[
            pl.BlockSpec((nblk, d), lambda i: (i, 0)),
            pl.BlockSpec((eblk, de), lambda i: (i, 0)),
            pl.BlockSpec((eblk, 1), lambda i: (i, 0)),
            pl.BlockSpec((d, h), full), pl.BlockSpec((h,), full1),
            pl.BlockSpec((h, 1), full), pl.BlockSpec((1,), full1),
            pl.BlockSpec((d, h), full), pl.BlockSpec((h,), full1),
            pl.BlockSpec((h, 1), full), pl.BlockSpec((1,), full1),
            pl.BlockSpec((de, h), full), pl.BlockSpec((h,), full1),
            pl.BlockSpec((h, 1), full), pl.BlockSpec((1,), full1),
        ],
        out_specs=[
            pl.BlockSpec((nblk, 2), lambda i: (i, 0)),
            pl.BlockSpec((eblk, 1), lambda i: (i, 0)),
        ],
        out_shape=[
            jax.ShapeDtypeStruct((n, 2), jnp.float32),
            jax.ShapeDtypeStruct((e, 1), jnp.float32),
        ],
    )(node_emb, relation_emb, g2,
      src_w1, src_b1, src_w2, src_b2,
      dst_w1, dst_b1, dst_w2, dst_b2,
      edge_w1, edge_b1, edge_w2, edge_b2)


def _sc_gather_gate(scores_flat, ei_flat, ge):
    n2 = scores_flat.shape[0]
    e = ei_flat.shape[0] // 2
    ch = e // NW
    full = (ch // LANES) * LANES
    tail = ch - full
    mesh = plsc.VectorSubcoreMesh(
        core_axis_name="c", subcore_axis_name="s",
        num_cores=NC, num_subcores=NS)

    @functools.partial(
        pl.kernel,
        out_type=[
            jax.ShapeDtypeStruct((e,), jnp.float32),
            jax.ShapeDtypeStruct((NW * LANES,), jnp.float32),
        ],
        mesh=mesh,
        compiler_params=pltpu.CompilerParams(needs_layout_passes=False),
        scratch_types=[
            pltpu.VMEM((n2,), jnp.float32),
            pltpu.VMEM((ch,), jnp.int32),
            pltpu.VMEM((ch,), jnp.int32),
            pltpu.VMEM((ch,), jnp.float32),
            pltpu.VMEM((ch,), jnp.float32),
            pltpu.VMEM((LANES,), jnp.float32),
        ],
    )
    def sc_kernel(scores_hbm, ei_hbm, ge_hbm, aug_hbm, part_hbm,
                  table_v, src_v, dst_v, ge_v, aug_v, acc_v):
        c = lax.axis_index("c")
        s = lax.axis_index("s")
        wid = s * NC + c
        base = wid * ch
        pltpu.sync_copy(scores_hbm, table_v)
        pltpu.sync_copy(ei_hbm.at[pl.ds(base, ch)], src_v)
        pltpu.sync_copy(ei_hbm.at[pl.ds(e + base, ch)], dst_v)
        pltpu.sync_copy(ge_hbm.at[pl.ds(base, ch)], ge_v)

        def gate16(off):
            sidx = src_v[pl.ds(off, LANES)]
            didx = dst_v[pl.ds(off, LANES)]
            ws = plsc.load_gather(table_v, [sidx * 2])
            wd = plsc.load_gather(table_v, [didx * 2 + 1])
            x = (ws + wd + ge_v[pl.ds(off, LANES)]) * (1.0 / TEMPERATURE)
            return 1.0 / (1.0 + jnp.exp(-x))

        def body(i, acc):
            off = i * LANES
            a = gate16(off)
            aug_v[pl.ds(off, LANES)] = a
            return acc + a

        acc = lax.fori_loop(0, full // LANES, body,
                            jnp.zeros((LANES,), jnp.float32))
        if tail:
            off = ch - LANES
            a = gate16(off)
            aug_v[pl.ds(off, LANES)] = a
            lane = lax.iota(jnp.int32, LANES)
            acc = acc + jnp.where(lane >= (LANES - tail), a, 0.0)
        acc_v[...] = acc
        pltpu.sync_copy(aug_v, aug_hbm.at[pl.ds(base, ch)])
        pltpu.sync_copy(acc_v, part_hbm.at[pl.ds(wid * LANES, LANES)])

    return sc_kernel(scores_flat, ei_flat, ge)


def kernel(node_emb, edge_index, relation_emb,
           src_w1, src_b1, src_w2, src_b2,
           dst_w1, dst_b1, dst_w2, dst_b2,
           edge_w1, edge_b1, edge_w2, edge_b2):
    n = node_emb.shape[0]
    e = edge_index.shape[1]

    u = jax.random.uniform(jax.random.key(12345), (e,), jnp.float32)
    eps = (BIAS - (1.0 - BIAS)) * u + (1.0 - BIAS)
    g2 = (jnp.log(eps) - jnp.log(1.0 - eps)).reshape(e, 1)

    scores, ge2 = _tc_mlps(node_emb, relation_emb, g2,
                           src_w1, src_b1, src_w2, src_b2,
                           dst_w1, dst_b1, dst_w2, dst_b2,
                           edge_w1, edge_b1, edge_w2, edge_b2)

    aug, partials = _sc_gather_gate(scores.reshape(2 * n),
                                    edge_index.reshape(2 * e),
                                    ge2.reshape(e))

    reg = 1.0 - jnp.sum(partials) / e
    return (reg, aug)


# speedup vs baseline: 3.8674x; 3.8674x over previous
import functools

import jax
import jax.numpy as jnp
from jax import lax
from jax.experimental import pallas as pl
from jax.experimental.pallas import tpu as pltpu
from jax.experimental.pallas import tpu_sc as plsc

TEMPERATURE = 0.5
BIAS = 0.0001

NC = 2
NS = 16
NW = NC * NS
LANES = 16

GRID = 10


def _tc_body(x_ref, rel_ref,
             sw1_ref, sb1_ref, sw2_ref, sb2_ref,
             dw1_ref, db1_ref, dw2_ref, db2_ref,
             ew1_ref, eb1_ref, ew2_ref, eb2_ref,
             scores_ref, ge_ref):
    eblk = ge_ref.shape[2]
    x = x_ref[...]
    hs = jnp.maximum(
        jnp.dot(x, sw1_ref[...], preferred_element_type=jnp.float32)
        + sb1_ref[...], 0.0)
    ss = jnp.dot(hs, sw2_ref[...], preferred_element_type=jnp.float32) + sb2_ref[...]
    hd = jnp.maximum(
        jnp.dot(x, dw1_ref[...], preferred_element_type=jnp.float32)
        + db1_ref[...], 0.0)
    sd = jnp.dot(hd, dw2_ref[...], preferred_element_type=jnp.float32) + db2_ref[...]
    scores_ref[...] = jnp.concatenate([ss, sd], axis=1)

    he = jnp.maximum(
        jnp.dot(rel_ref[...], ew1_ref[...], preferred_element_type=jnp.float32)
        + eb1_ref[...], 0.0)
    er = lax.dot_general(ew2_ref[...], he, (((0,), (1,)), ((), ())),
                         preferred_element_type=jnp.float32)
    ge_ref[...] = (er + eb2_ref[...]).reshape(1, 1, eblk)


def _tc_mlps(node_emb, relation_emb,
             src_w1, src_b1, src_w2, src_b2,
             dst_w1, dst_b1, dst_w2, dst_b2,
             edge_w1, edge_b1, edge_w2, edge_b2):
    n, d = node_emb.shape
    e, de = relation_emb.shape
    h = src_w1.shape[1]
    nblk = n // GRID
    eblk = e // GRID
    full = lambda i: (0, 0)
    full1 = lambda i: (0,)
    return pl.pallas_call(
        _tc_body,
        grid=(GRID,),
        in_specs=[
            pl.BlockSpec((nblk, d), lambda i: (i, 0)),
            pl.BlockSpec((eblk, de), lambda i: (i, 0)),
            pl.BlockSpec((d, h), full), pl.BlockSpec((h,), full1),
            pl.BlockSpec((h, 1), full), pl.BlockSpec((1,), full1),
            pl.BlockSpec((d, h), full), pl.BlockSpec((h,), full1),
            pl.BlockSpec((h, 1), full), pl.BlockSpec((1,), full1),
            pl.BlockSpec((de, h), full), pl.BlockSpec((h,), full1),
            pl.BlockSpec((h, 1), full), pl.BlockSpec((1,), full1),
        ],
        out_specs=[
            pl.BlockSpec((nblk, 2), lambda i: (i, 0)),
            pl.BlockSpec((1, 1, eblk), lambda i: (i, 0, 0)),
        ],
        out_shape=[
            jax.ShapeDtypeStruct((n, 2), jnp.float32),
            jax.ShapeDtypeStruct((GRID, 1, eblk), jnp.float32),
        ],
    )(node_emb, relation_emb,
      src_w1, src_b1, src_w2, src_b2,
      dst_w1, dst_b1, dst_w2, dst_b2,
      edge_w1, edge_b1, edge_w2, edge_b2)


def _sc_gather_gate(scores_flat, ei_flat, ge, g):
    n2 = scores_flat.shape[0]
    e = ei_flat.shape[0] // 2
    ch = e // NW
    full = (ch // LANES) * LANES
    tail = ch - full
    mesh = plsc.VectorSubcoreMesh(
        core_axis_name="c", subcore_axis_name="s",
        num_cores=NC, num_subcores=NS)

    @functools.partial(
        pl.kernel,
        out_type=[
            jax.ShapeDtypeStruct((e,), jnp.float32),
            jax.ShapeDtypeStruct((NW * LANES,), jnp.float32),
        ],
        mesh=mesh,
        compiler_params=pltpu.CompilerParams(needs_layout_passes=False),
        scratch_types=[
            pltpu.VMEM((n2,), jnp.float32),
            pltpu.VMEM((ch,), jnp.int32),
            pltpu.VMEM((ch,), jnp.int32),
            pltpu.VMEM((ch,), jnp.float32),
            pltpu.VMEM((ch,), jnp.float32),
            pltpu.VMEM((ch,), jnp.float32),
            pltpu.VMEM((LANES,), jnp.float32),
        ],
    )
    def sc_kernel(scores_hbm, ei_hbm, ge_hbm, g_hbm, aug_hbm, part_hbm,
                  table_v, src_v, dst_v, ge_v, g_v, aug_v, acc_v):
        c = lax.axis_index("c")
        s = lax.axis_index("s")
        wid = s * NC + c
        base = wid * ch
        pltpu.sync_copy(scores_hbm, table_v)
        pltpu.sync_copy(ei_hbm.at[pl.ds(base, ch)], src_v)
        pltpu.sync_copy(ei_hbm.at[pl.ds(e + base, ch)], dst_v)
        pltpu.sync_copy(ge_hbm.at[pl.ds(base, ch)], ge_v)
        pltpu.sync_copy(g_hbm.at[pl.ds(base, ch)], g_v)

        def gate16(off):
            sidx = src_v[pl.ds(off, LANES)]
            didx = dst_v[pl.ds(off, LANES)]
            ws = plsc.load_gather(table_v, [sidx * 2])
            wd = plsc.load_gather(table_v, [didx * 2 + 1])
            x = (ws + wd + ge_v[pl.ds(off, LANES)] + g_v[pl.ds(off, LANES)]) \
                * (1.0 / TEMPERATURE)
            return 1.0 / (1.0 + jnp.exp(-x))

        def body(i, acc):
            off = i * LANES
            a = gate16(off)
            aug_v[pl.ds(off, LANES)] = a
            return acc + a

        acc = lax.fori_loop(0, full // LANES, body,
                            jnp.zeros((LANES,), jnp.float32))
        if tail:
            off = ch - LANES
            a = gate16(off)
            aug_v[pl.ds(off, LANES)] = a
            lane = lax.iota(jnp.int32, LANES)
            acc = acc + jnp.where(lane >= (LANES - tail), a, 0.0)
        acc_v[...] = acc
        pltpu.sync_copy(aug_v, aug_hbm.at[pl.ds(base, ch)])
        pltpu.sync_copy(acc_v, part_hbm.at[pl.ds(wid * LANES, LANES)])

    return sc_kernel(scores_flat, ei_flat, ge, g)


def kernel(node_emb, edge_index, relation_emb,
           src_w1, src_b1, src_w2, src_b2,
           dst_w1, dst_b1, dst_w2, dst_b2,
           edge_w1, edge_b1, edge_w2, edge_b2):
    n = node_emb.shape[0]
    e = edge_index.shape[1]

    u = jax.random.uniform(jax.random.key(12345), (e,), jnp.float32)
    eps = (BIAS - (1.0 - BIAS)) * u + (1.0 - BIAS)
    g = jnp.log(eps) - jnp.log(1.0 - eps)

    scores, ge3 = _tc_mlps(node_emb, relation_emb,
                           src_w1, src_b1, src_w2, src_b2,
                           dst_w1, dst_b1, dst_w2, dst_b2,
                           edge_w1, edge_b1, edge_w2, edge_b2)

    aug, partials = _sc_gather_gate(scores.reshape(2 * n),
                                    edge_index.reshape(2 * e),
                                    ge3.reshape(e), g)

    reg = 1.0 - jnp.sum(partials) / e
    return (reg, aug)
